# baseline (device time: 22772 ns/iter reference)
import jax
import jax.numpy as jnp
from jax import lax
from jax.experimental import pallas as pl
from jax.experimental.pallas import tpu as pltpu

N_DEV = 32
TAPS = 4
HALO = TAPS - 1


def kernel(x, k):
    b, s, c = x.shape

    def body(x_ref, k_ref, out_ref, halo_ref, send_ref, send_sem, recv_sem):
        my = lax.axis_index("i")
        left = lax.rem(my + N_DEV - 1, N_DEV)
        right = lax.rem(my + 1, N_DEV)

        barrier = pltpu.get_barrier_semaphore()
        for nbr in (left, right):
            pl.semaphore_signal(
                barrier, inc=1,
                device_id=(nbr,), device_id_type=pl.DeviceIdType.MESH,
            )
        pl.semaphore_wait(barrier, 2)

        send_ref[...] = x_ref[:, s - HALO:s, :]
        rdma = pltpu.make_async_remote_copy(
            src_ref=send_ref,
            dst_ref=halo_ref,
            send_sem=send_sem,
            recv_sem=recv_sem,
            device_id=(right,),
            device_id_type=pl.DeviceIdType.MESH,
        )
        rdma.start()
        rdma.wait()

        @pl.when(my == 0)
        def _():
            halo_ref[...] = jnp.zeros_like(halo_ref)

        pad = jnp.concatenate([halo_ref[...], x_ref[...]], axis=1)
        kv = k_ref[...]
        acc = pad[:, 0:s, :] * kv[0][None, None, :]
        for t in range(1, TAPS):
            acc += pad[:, t:t + s, :] * kv[t][None, None, :]
        out_ref[...] = acc * jax.nn.sigmoid(acc)

    return pl.pallas_call(
        body,
        out_shape=jax.ShapeDtypeStruct((b, s, c), jnp.float32),
        in_specs=[
            pl.BlockSpec(memory_space=pltpu.VMEM),
            pl.BlockSpec(memory_space=pltpu.VMEM),
        ],
        out_specs=pl.BlockSpec(memory_space=pltpu.VMEM),
        scratch_shapes=[
            pltpu.VMEM((b, HALO, c), jnp.float32),
            pltpu.VMEM((b, HALO, c), jnp.float32),
            pltpu.SemaphoreType.DMA,
            pltpu.SemaphoreType.DMA,
        ],
        compiler_params=pltpu.CompilerParams(collective_id=0),
    )(x, k)


# device time: 16427 ns/iter; 1.3863x vs baseline; 1.3863x over previous
import jax
import jax.numpy as jnp
from jax import lax
from jax.experimental import pallas as pl
from jax.experimental.pallas import tpu as pltpu

N_DEV = 32
TAPS = 4
HALO = TAPS - 1


def kernel(x, k):
    b, s, c = x.shape

    def body(x_ref, k_ref, out_ref, halo_ref, send_ref, send_sem, recv_sem):
        my = lax.axis_index("i")
        left = lax.rem(my + N_DEV - 1, N_DEV)
        right = lax.rem(my + 1, N_DEV)

        barrier = pltpu.get_barrier_semaphore()
        for nbr in (left, right):
            pl.semaphore_signal(
                barrier, inc=1,
                device_id=(nbr,), device_id_type=pl.DeviceIdType.MESH,
            )
        pl.semaphore_wait(barrier, 2)

        send_ref[...] = x_ref[:, s - HALO:s, :]
        rdma = pltpu.make_async_remote_copy(
            src_ref=send_ref,
            dst_ref=halo_ref,
            send_sem=send_sem,
            recv_sem=recv_sem,
            device_id=(right,),
            device_id_type=pl.DeviceIdType.MESH,
        )
        rdma.start()

        xv = x_ref[...]
        kv = k_ref[...]
        acc = xv[:, 0:s - HALO, :] * kv[0][None, None, :]
        for t in range(1, TAPS):
            acc += xv[:, t:t + s - HALO, :] * kv[t][None, None, :]
        out_ref[:, HALO:s, :] = acc * jax.nn.sigmoid(acc)

        rdma.wait_recv()

        @pl.when(my == 0)
        def _():
            halo_ref[...] = jnp.zeros_like(halo_ref)

        pad = jnp.concatenate([halo_ref[...], xv[:, 0:HALO, :]], axis=1)
        accb = pad[:, 0:HALO, :] * kv[0][None, None, :]
        for t in range(1, TAPS):
            accb += pad[:, t:t + HALO, :] * kv[t][None, None, :]
        out_ref[:, 0:HALO, :] = accb * jax.nn.sigmoid(accb)

        rdma.wait_send()

    return pl.pallas_call(
        body,
        out_shape=jax.ShapeDtypeStruct((b, s, c), jnp.float32),
        in_specs=[
            pl.BlockSpec(memory_space=pltpu.VMEM),
            pl.BlockSpec(memory_space=pltpu.VMEM),
        ],
        out_specs=pl.BlockSpec(memory_space=pltpu.VMEM),
        scratch_shapes=[
            pltpu.VMEM((b, HALO, c), jnp.float32),
            pltpu.VMEM((b, HALO, c), jnp.float32),
            pltpu.SemaphoreType.DMA,
            pltpu.SemaphoreType.DMA,
        ],
        compiler_params=pltpu.CompilerParams(collective_id=0),
    )(x, k)
